# Initial kernel scaffold; baseline (speedup 1.0000x reference)
import jax
import jax.numpy as jnp
from jax.experimental import pallas as pl


def kernel(x, indices):
    raise NotImplementedError("write your pallas kernel here")


# speedup vs baseline: 2.4041x; 2.4041x over previous
import dataclasses
import functools

import jax
import jax.numpy as jnp
from jax import lax
from jax.experimental import pallas as pl
from jax.experimental.pallas import tpu as pltpu
from jax.experimental.pallas import tpu_sc as plsc

N, C = 64, 768
CELLS_IN = 16
CELLS_OUT = 120
R = N * C
NUM_WORKERS = 32
RPW = R // NUM_WORKERS
CR = 128
NCHUNK = RPW // CR
LANES = 16


def _unpool_body(x_hbm, idx_hbm, out_hbm, xbuf, ibuf, obuf, sem_in, sem_out):
    wid = lax.axis_index("s") * 2 + lax.axis_index("c")
    wbase = wid * RPW

    lanes = lax.iota(jnp.int32, LANES)

    @pl.loop(0, NCHUNK)
    def _chunk(t):
        rowbase = wbase + t * CR
        pltpu.sync_copy(x_hbm.at[pl.ds(rowbase * CELLS_IN, CR * CELLS_IN)], xbuf)
        pltpu.sync_copy(idx_hbm.at[pl.ds(rowbase * CELLS_IN, CR * CELLS_IN)], ibuf)

        zeros = jnp.zeros((LANES,), jnp.float32)

        @pl.loop(0, CR * CELLS_OUT, step=LANES)
        def _zero(k):
            obuf[pl.ds(k, LANES)] = zeros

        @pl.loop(0, CR, step=LANES)
        def _group(g):
            rows = lanes + g
            src = rows * CELLS_IN
            dst = rows * CELLS_OUT
            for i in range(CELLS_IN):
                vals = plsc.load_gather(xbuf, [src + i])
                idxs = plsc.load_gather(ibuf, [src + i])
                plsc.store_scatter(obuf, [dst + idxs], vals)

        pltpu.sync_copy(obuf, out_hbm.at[pl.ds(rowbase * CELLS_OUT, CR * CELLS_OUT)])


@jax.jit
def _unpool(xf, idxf):
    mesh = plsc.VectorSubcoreMesh(core_axis_name="c", subcore_axis_name="s")
    return pl.kernel(
        _unpool_body,
        out_type=jax.ShapeDtypeStruct((R * CELLS_OUT,), jnp.float32),
        mesh=mesh,
        scratch_types=[
            pltpu.VMEM((CR * CELLS_IN,), jnp.float32),
            pltpu.VMEM((CR * CELLS_IN,), jnp.int32),
            pltpu.VMEM((CR * CELLS_OUT,), jnp.float32),
            pltpu.SemaphoreType.DMA,
            pltpu.SemaphoreType.DMA,
        ],
        compiler_params=pltpu.CompilerParams(needs_layout_passes=False),
    )(xf, idxf)


def kernel(x, indices):
    xf = x.reshape(R * CELLS_IN)
    idxf = indices.astype(jnp.int32).reshape(R * CELLS_IN)
    out = _unpool(xf, idxf)
    return out.reshape(N, C, 4, 5, 6)


# speedup vs baseline: 32.2695x; 13.4229x over previous
import jax
import jax.numpy as jnp
import numpy as np
from jax import lax
from jax.experimental import pallas as pl
from jax.experimental.pallas import tpu as pltpu
from jax.experimental.pallas import tpu_sc as plsc

BN, BC = 64, 768
CELLS_IN = 16
CELLS_OUT = 120
NUM_WORKERS = 32
N_PER_W = BN // NUM_WORKERS
CB = 128
NCC = BC // CB
LANES = 16

_PERM = np.zeros(128, dtype=np.int32)
for _v in range(CELLS_OUT):
    _d, _h, _w = _v // 30, (_v // 6) % 5, _v % 6
    _PERM[_v] = _h * 24 + _w * 4 + _d


def _unpool_body(x_hbm, idx_hbm, ptab_hbm, out_hbm, xbuf, ibuf, obuf, ptab):
    wid = lax.axis_index("s") * 2 + lax.axis_index("c")
    pltpu.sync_copy(ptab_hbm, ptab)

    lanes = lax.iota(jnp.int32, LANES)
    zeros = jnp.zeros((LANES,), jnp.float32)

    @pl.loop(0, N_PER_W)
    def _n(tn):
        n = wid * N_PER_W + tn

        @pl.loop(0, NCC)
        def _cc(cc):
            c0 = cc * CB
            pltpu.sync_copy(x_hbm.at[pl.ds(n * CELLS_IN, CELLS_IN), pl.ds(c0, CB)], xbuf)
            pltpu.sync_copy(idx_hbm.at[pl.ds(n * CELLS_IN, CELLS_IN), pl.ds(c0, CB)], ibuf)

            @pl.loop(0, CELLS_OUT)
            def _zero(r):
                for k in range(CB // LANES):
                    obuf[r, pl.ds(k * LANES, LANES)] = zeros

            @pl.loop(0, CELLS_IN)
            def _cell(p):
                for k in range(CB // LANES):
                    cols = lanes + k * LANES
                    vals = xbuf[p, pl.ds(k * LANES, LANES)]
                    idxv = ibuf[p, pl.ds(k * LANES, LANES)]
                    rows = plsc.load_gather(ptab, [idxv])
                    plsc.store_scatter(obuf, [rows, cols], vals)

            pltpu.sync_copy(obuf, out_hbm.at[pl.ds(n * CELLS_OUT, CELLS_OUT), pl.ds(c0, CB)])


@jax.jit
def _unpool(x2, i2, ptab):
    mesh = plsc.VectorSubcoreMesh(core_axis_name="c", subcore_axis_name="s")
    return pl.kernel(
        _unpool_body,
        out_type=jax.ShapeDtypeStruct((BN * CELLS_OUT, BC), jnp.float32),
        mesh=mesh,
        scratch_types=[
            pltpu.VMEM((CELLS_IN, CB), jnp.float32),
            pltpu.VMEM((CELLS_IN, CB), jnp.int32),
            pltpu.VMEM((CELLS_OUT, CB), jnp.float32),
            pltpu.VMEM((128,), jnp.int32),
        ],
        compiler_params=pltpu.CompilerParams(needs_layout_passes=False),
    )(x2, i2, ptab)


def kernel(x, indices):
    x2 = jnp.transpose(x, (0, 2, 3, 4, 1)).reshape(BN * CELLS_IN, BC)
    i2 = jnp.transpose(indices.astype(jnp.int32), (0, 2, 3, 4, 1)).reshape(BN * CELLS_IN, BC)
    out2 = _unpool(x2, i2, jnp.asarray(_PERM))
    out_t = out2.reshape(BN, 5, 6, 4, BC)
    return jnp.transpose(out_t, (0, 4, 3, 1, 2))


# speedup vs baseline: 38.0687x; 1.1797x over previous
import jax
import jax.numpy as jnp
import numpy as np
from jax import lax
from jax.experimental import pallas as pl
from jax.experimental.pallas import tpu as pltpu
from jax.experimental.pallas import tpu_sc as plsc

BN, BC = 64, 768
CELLS_IN = 16
CELLS_OUT = 120
NUM_WORKERS = 32
N_PER_W = BN // NUM_WORKERS
CB = 128
NCC = BC // CB
LANES = 16

_TABS = np.zeros(384, dtype=np.int32)
for _v in range(CELLS_OUT):
    _TABS[_v] = (_v // 6) % 5
    _TABS[128 + _v] = _v % 6
    _TABS[256 + _v] = _v // 30


def _unpool_body(x_hbm, idx_hbm, ptab_hbm, out_hbm, xbuf, ibuf, obuf, th, tw, td):
    wid = lax.axis_index("s") * 2 + lax.axis_index("c")
    pltpu.sync_copy(ptab_hbm.at[pl.ds(0, 128)], th)
    pltpu.sync_copy(ptab_hbm.at[pl.ds(128, 128)], tw)
    pltpu.sync_copy(ptab_hbm.at[pl.ds(256, 128)], td)

    lanes = lax.iota(jnp.int32, LANES)
    zeros = jnp.zeros((LANES,), jnp.float32)

    @pl.loop(0, N_PER_W)
    def _n(tn):
        n = wid * N_PER_W + tn

        @pl.loop(0, NCC)
        def _cc(cc):
            c0 = cc * CB
            pltpu.sync_copy(x_hbm.at[n, :, :, :, pl.ds(c0, CB)], xbuf)
            pltpu.sync_copy(idx_hbm.at[n, :, :, :, pl.ds(c0, CB)], ibuf)

            @pl.loop(0, 5)
            def _zero(h):
                for w in range(6):
                    for d in range(4):
                        for k in range(CB // LANES):
                            obuf[h, w, d, pl.ds(k * LANES, LANES)] = zeros

            for k in range(CB // LANES):
                cols = lanes + k * LANES
                for p in range(CELLS_IN):
                    z, y, w = p // 8, (p // 4) % 2, p % 4
                    vals = xbuf[z, y, w, pl.ds(k * LANES, LANES)]
                    idxv = ibuf[z, y, w, pl.ds(k * LANES, LANES)]
                    hv = plsc.load_gather(th, [idxv])
                    wv = plsc.load_gather(tw, [idxv])
                    dv = plsc.load_gather(td, [idxv])
                    plsc.store_scatter(obuf, [hv, wv, dv, cols], vals)

            pltpu.sync_copy(obuf, out_hbm.at[n, :, :, :, pl.ds(c0, CB)])


@jax.jit
def _unpool(x5, i5, ptab):
    mesh = plsc.VectorSubcoreMesh(core_axis_name="c", subcore_axis_name="s")
    return pl.kernel(
        _unpool_body,
        out_type=jax.ShapeDtypeStruct((BN, 5, 6, 4, BC), jnp.float32),
        mesh=mesh,
        scratch_types=[
            pltpu.VMEM((2, 2, 4, CB), jnp.float32),
            pltpu.VMEM((2, 2, 4, CB), jnp.int32),
            pltpu.VMEM((5, 6, 4, CB), jnp.float32),
            pltpu.VMEM((128,), jnp.int32),
            pltpu.VMEM((128,), jnp.int32),
            pltpu.VMEM((128,), jnp.int32),
        ],
        compiler_params=pltpu.CompilerParams(needs_layout_passes=False),
    )(x5, i5, ptab)


def kernel(x, indices):
    x5 = jnp.transpose(x, (0, 2, 3, 4, 1))
    i5 = jnp.transpose(indices.astype(jnp.int32), (0, 2, 3, 4, 1))
    out5 = _unpool(x5, i5, jnp.asarray(_TABS))
    return jnp.transpose(out5, (0, 4, 3, 1, 2))


# speedup vs baseline: 64.7804x; 1.7017x over previous
import jax
import jax.numpy as jnp
import numpy as np
from jax import lax
from jax.experimental import pallas as pl
from jax.experimental.pallas import tpu as pltpu
from jax.experimental.pallas import tpu_sc as plsc

BN, BC = 64, 768
CELLS_IN = 16
NUM_WORKERS = 32
N_PER_W = BN // NUM_WORKERS
CB = 128
NCC = BC // CB
LANES = 16

_TAB = np.zeros(128, dtype=np.int32)
for _v in range(120):
    _TAB[_v] = (_v % 30) * 4 + _v // 30


def _unpool_body(x_hbm, idx_hbm, ptab_hbm, out_hbm,
                 xb0, xb1, ib0, ib1, ob0, ob1, tbd,
                 sx0, sx1, si0, si1, so0, so1):
    wid = lax.axis_index("s") * 2 + lax.axis_index("c")
    pltpu.sync_copy(ptab_hbm, tbd)

    lanes = lax.iota(jnp.int32, LANES)
    zeros = jnp.zeros((LANES,), jnp.float32)
    bufs = ((xb0, ib0, ob0, sx0, si0, so0, wid * N_PER_W),
            (xb1, ib1, ob1, sx1, si1, so1, wid * N_PER_W + 1))

    def issue_in(cc, b):
        xb, ib, _, sx, si, _, n = bufs[b]
        c0 = cc * CB
        pltpu.async_copy(x_hbm.at[n, :, :, :, pl.ds(c0, CB)], xb, sx)
        pltpu.async_copy(idx_hbm.at[n, :, :, :, pl.ds(c0, CB)], ib, si)

    for b in range(2):
        issue_in(0, b)

    @pl.loop(0, NCC)
    def _cc(cc):
        c0 = cc * CB
        for b in range(2):
            xb, ib, ob, sx, si, so, n = bufs[b]

            @pl.when(cc > 0)
            def _():
                pltpu.make_async_copy(ob, out_hbm.at[n, :, :, pl.ds(c0, CB)], so).wait()

            @pl.loop(0, 30)
            def _zero(r):
                for d in range(4):
                    for k in range(CB // LANES):
                        ob[r, d, pl.ds(k * LANES, LANES)] = zeros

            pltpu.make_async_copy(x_hbm.at[n, :, :, :, pl.ds(c0, CB)], xb, sx).wait()
            pltpu.make_async_copy(idx_hbm.at[n, :, :, :, pl.ds(c0, CB)], ib, si).wait()

            for k in range(CB // LANES):
                cols = lanes + k * LANES
                for p in range(CELLS_IN):
                    z, y, w = p // 8, (p // 4) % 2, p % 4
                    vals = xb[z, y, w, pl.ds(k * LANES, LANES)]
                    idxv = ib[z, y, w, pl.ds(k * LANES, LANES)]
                    bd = plsc.load_gather(tbd, [idxv])
                    bv = jax.lax.shift_right_logical(bd, 2)
                    dv = jax.lax.bitwise_and(bd, 3)
                    plsc.store_scatter(ob, [bv, dv, cols], vals)

            @pl.when(cc < NCC - 1)
            def _():
                issue_in(cc + 1, b)

            pltpu.async_copy(ob, out_hbm.at[n, :, :, pl.ds(c0, CB)], so)

    for b in range(2):
        xb, ib, ob, sx, si, so, n = bufs[b]
        pltpu.make_async_copy(ob, out_hbm.at[n, :, :, pl.ds((NCC - 1) * CB, CB)], so).wait()


@jax.jit
def _unpool(x5, i5, ptab):
    mesh = plsc.VectorSubcoreMesh(core_axis_name="c", subcore_axis_name="s")
    return pl.kernel(
        _unpool_body,
        out_type=jax.ShapeDtypeStruct((BN, 30, 4, BC), jnp.float32),
        mesh=mesh,
        scratch_types=[
            pltpu.VMEM((2, 2, 4, CB), jnp.float32),
            pltpu.VMEM((2, 2, 4, CB), jnp.float32),
            pltpu.VMEM((2, 2, 4, CB), jnp.int32),
            pltpu.VMEM((2, 2, 4, CB), jnp.int32),
            pltpu.VMEM((30, 4, CB), jnp.float32),
            pltpu.VMEM((30, 4, CB), jnp.float32),
            pltpu.VMEM((128,), jnp.int32),
            pltpu.SemaphoreType.DMA,
            pltpu.SemaphoreType.DMA,
            pltpu.SemaphoreType.DMA,
            pltpu.SemaphoreType.DMA,
            pltpu.SemaphoreType.DMA,
            pltpu.SemaphoreType.DMA,
        ],
        compiler_params=pltpu.CompilerParams(needs_layout_passes=False),
    )(x5, i5, ptab)


def kernel(x, indices):
    x5 = jnp.transpose(x, (0, 2, 3, 4, 1))
    i5 = jnp.transpose(indices.astype(jnp.int32), (0, 2, 3, 4, 1))
    out4 = _unpool(x5, i5, jnp.asarray(_TAB))
    out5 = out4.reshape(BN, 5, 6, 4, BC)
    return jnp.transpose(out5, (0, 4, 3, 1, 2))
